# Initial kernel scaffold; baseline (speedup 1.0000x reference)
import jax
import jax.numpy as jnp
from jax.experimental import pallas as pl


def kernel(x, edge_index, W1, W2):
    raise NotImplementedError("write your pallas kernel here")


# speedup vs baseline: 13.3508x; 13.3508x over previous
import functools
import math

import jax
import jax.numpy as jnp
from jax import lax
from jax.experimental import pallas as pl
from jax.experimental.pallas import tpu as pltpu
from jax.experimental.pallas import tpu_sc as plsc

ALPHA = 0.5
THETA = 1.0

_NC = 2
_NS = 16
_NW = _NC * _NS
_LANES = 16
_K = 80
_ZR = 40


def _deg_body(N, E, dst_hbm, ones_hbm, zer_hbm, out_hbm, idx_d, onev, zbuf, acc):
    c = lax.axis_index("c")
    s = lax.axis_index("s")
    wid = c * _NS + s
    n_chunks = (E // _NW) // _K
    nch = N // _ZR

    pltpu.sync_copy(dst_hbm.at[wid], idx_d)
    pltpu.sync_copy(ones_hbm, onev)
    pltpu.sync_copy(zer_hbm, zbuf)

    def zc_body(m, carry):
        ch = m * _NS + s

        @pl.when(ch < nch)
        def _():
            pltpu.sync_copy(zbuf, acc.at[pl.ds(ch * _ZR, _ZR)])

        return carry

    lax.fori_loop(0, pl.cdiv(nch, _NS), zc_body, 0)
    plsc.subcore_barrier()

    def body(j, carry):
        pltpu.sync_copy(onev, acc.at[idx_d.at[j]], add=True)
        return carry

    lax.fori_loop(0, n_chunks, body, 0)
    plsc.subcore_barrier()

    def out_body(m, carry):
        ch = m * _NS + s

        @pl.when(ch < nch)
        def _():
            pltpu.sync_copy(acc.at[pl.ds(ch * _ZR, _ZR)], zbuf)
            pltpu.sync_copy(zbuf, out_hbm.at[c, pl.ds(ch * _ZR, _ZR)])

        return carry

    lax.fori_loop(0, pl.cdiv(nch, _NS), out_body, 0)


def _make_deg_kernel(N, D, E):
    n_chunks = (E // _NW) // _K
    mesh = plsc.VectorSubcoreMesh(core_axis_name="c", subcore_axis_name="s")
    return pl.kernel(
        functools.partial(_deg_body, N, E),
        mesh=mesh,
        out_type=jax.ShapeDtypeStruct((_NC, N, D), jnp.float32),
        scratch_types=[
            pltpu.VMEM((n_chunks, _K), jnp.int32),
            pltpu.VMEM((_K, D), jnp.float32),
            pltpu.VMEM((_ZR, D), jnp.float32),
            pltpu.VMEM_SHARED((N, D), jnp.float32),
        ],
    )


def _agg_body(N, E, g_hbm, src_hbm, dst_hbm, zer_hbm, out_hbm,
              idx_s, idx_d, rows, zbuf, acc, sem):
    c = lax.axis_index("c")
    s = lax.axis_index("s")
    wid = c * _NS + s
    n_chunks = (E // _NW) // _K

    pltpu.sync_copy(src_hbm.at[wid], idx_s)
    pltpu.sync_copy(dst_hbm.at[wid], idx_d)

    pltpu.sync_copy(zer_hbm, zbuf)
    nch = N // _ZR

    def zc_body(m, carry):
        ch = m * _NS + s

        @pl.when(ch < nch)
        def _():
            pltpu.sync_copy(zbuf, acc.at[pl.ds(ch * _ZR, _ZR)])

        return carry

    lax.fori_loop(0, pl.cdiv(nch, _NS), zc_body, 0)
    plsc.subcore_barrier()

    def body(j, carry):
        pltpu.async_copy(g_hbm.at[idx_s.at[j]], rows, sem).wait()
        pltpu.sync_copy(rows, acc.at[idx_d.at[j]], add=True)
        return carry

    lax.fori_loop(0, n_chunks, body, 0)
    plsc.subcore_barrier()

    def out_body(m, carry):
        ch = m * _NS + s

        @pl.when(ch < nch)
        def _():
            pltpu.sync_copy(acc.at[pl.ds(ch * _ZR, _ZR)], zbuf)
            pltpu.sync_copy(zbuf, out_hbm.at[c, pl.ds(ch * _ZR, _ZR)])

        return carry

    lax.fori_loop(0, pl.cdiv(nch, _NS), out_body, 0)


def _make_agg_kernel(N, D, E):
    n_chunks = (E // _NW) // _K
    mesh = plsc.VectorSubcoreMesh(core_axis_name="c", subcore_axis_name="s")
    return pl.kernel(
        functools.partial(_agg_body, N, E),
        mesh=mesh,
        out_type=jax.ShapeDtypeStruct((_NC, N, D), jnp.float32),
        scratch_types=[
            pltpu.VMEM((n_chunks, _K), jnp.int32),
            pltpu.VMEM((n_chunks, _K), jnp.int32),
            pltpu.VMEM((_K, D), jnp.float32),
            pltpu.VMEM((_ZR, D), jnp.float32),
            pltpu.VMEM_SHARED((N, D), jnp.float32),
            pltpu.SemaphoreType.DMA,
        ],
    )


def _pre_body(betas, x_ref, w2_ref, degp_ref, g0_ref, dinv_ref, xw2_ref):
    x0 = jnp.maximum(x_ref[...], 0.0)
    deg = degp_ref[0][:, 0:1] + degp_ref[1][:, 0:1] + 1.0
    dinv = lax.rsqrt(deg)
    dinv_ref[...] = jnp.broadcast_to(dinv, x0.shape)
    g0_ref[...] = dinv * x0
    for i, b in enumerate(betas):
        m = jnp.dot(x0, w2_ref[i], preferred_element_type=jnp.float32)
        xw2_ref[i] = ALPHA * ((1.0 - b) * x0 + b * m)


def _make_pre(N, D, Lw, betas, R):
    return pl.pallas_call(
        functools.partial(_pre_body, betas),
        grid=(N // R,),
        in_specs=[
            pl.BlockSpec((R, D), lambda i: (i, 0)),
            pl.BlockSpec((Lw, D, D), lambda i: (0, 0, 0)),
            pl.BlockSpec((_NC, R, D), lambda i: (0, i, 0)),
        ],
        out_specs=[
            pl.BlockSpec((R, D), lambda i: (i, 0)),
            pl.BlockSpec((R, D), lambda i: (i, 0)),
            pl.BlockSpec((Lw, R, D), lambda i: (0, i, 0)),
        ],
        out_shape=[
            jax.ShapeDtypeStruct((N, D), jnp.float32),
            jax.ShapeDtypeStruct((N, D), jnp.float32),
            jax.ShapeDtypeStruct((Lw, N, D), jnp.float32),
        ],
    )


def _layer_body(beta, p_ref, g_ref, dinv_ref, xw2_ref, w1_ref, h_ref, g2_ref):
    dinv = dinv_ref[...]
    t = dinv * (p_ref[0] + p_ref[1] + g_ref[...])
    m = jnp.dot(t, w1_ref[...], preferred_element_type=jnp.float32)
    u = (1.0 - ALPHA) * ((1.0 - beta) * t + beta * m) + xw2_ref[...]
    h = jnp.maximum(u, 0.0)
    h_ref[...] = h
    g2_ref[...] = dinv * h


def _make_layer(N, D, beta, R):
    return pl.pallas_call(
        functools.partial(_layer_body, beta),
        grid=(N // R,),
        in_specs=[
            pl.BlockSpec((_NC, R, D), lambda i: (0, i, 0)),
            pl.BlockSpec((R, D), lambda i: (i, 0)),
            pl.BlockSpec((R, D), lambda i: (i, 0)),
            pl.BlockSpec((R, D), lambda i: (i, 0)),
            pl.BlockSpec((D, D), lambda i: (0, 0)),
        ],
        out_specs=[
            pl.BlockSpec((R, D), lambda i: (i, 0)),
            pl.BlockSpec((R, D), lambda i: (i, 0)),
        ],
        out_shape=[
            jax.ShapeDtypeStruct((N, D), jnp.float32),
            jax.ShapeDtypeStruct((N, D), jnp.float32),
        ],
    )


@jax.jit
def kernel(x, edge_index, W1, W2):
    N, D = x.shape
    E = edge_index.shape[1]
    Lw = W1.shape[0]
    betas = [math.log(THETA / (i + 1) + 1.0) for i in range(Lw)]
    R = 1000

    n_chunks = (E // _NW) // _K
    src3d = edge_index[0].reshape(_NW, n_chunks, _K)
    dst3d = edge_index[1].reshape(_NW, n_chunks, _K)
    zer = jnp.zeros((_ZR, D), jnp.float32)
    onesk = jnp.ones((_K, D), jnp.float32)

    degp = _make_deg_kernel(N, D, E)(dst3d, onesk, zer)
    g0, dinvb, xw2 = _make_pre(N, D, Lw, betas, R)(x, W2, degp)

    agg = _make_agg_kernel(N, D, E)
    g = g0
    h = None
    for i in range(Lw):
        part = agg(g, src3d, dst3d, zer)
        h, g = _make_layer(N, D, betas[i], R)(part, g, dinvb, xw2[i], W1[i])
    return h


# speedup vs baseline: 21.3243x; 1.5972x over previous
import functools
import math

import jax
import jax.numpy as jnp
from jax import lax
from jax.experimental import pallas as pl
from jax.experimental.pallas import tpu as pltpu
from jax.experimental.pallas import tpu_sc as plsc

ALPHA = 0.5
THETA = 1.0

_NC = 2
_NS = 16
_NW = _NC * _NS
_LANES = 16
_K = 100
_G = 16
_ZR = 40


def _deg_body(N, E, dst_hbm, ones_hbm, zer_hbm, out_hbm, idx_d, onev, zbuf, acc):
    c = lax.axis_index("c")
    s = lax.axis_index("s")
    wid = c * _NS + s
    n_chunks = (E // _NW) // _K
    nch = N // _ZR

    pltpu.sync_copy(dst_hbm.at[wid], idx_d)
    pltpu.sync_copy(ones_hbm, onev)
    pltpu.sync_copy(zer_hbm, zbuf)

    def zc_body(m, carry):
        ch = m * _NS + s

        @pl.when(ch < nch)
        def _():
            pltpu.sync_copy(zbuf, acc.at[pl.ds(ch * _ZR, _ZR)])

        return carry

    lax.fori_loop(0, pl.cdiv(nch, _NS), zc_body, 0)
    plsc.subcore_barrier()

    def body(j, carry):
        pltpu.sync_copy(onev, acc.at[idx_d.at[j]], add=True)
        return carry

    lax.fori_loop(0, n_chunks, body, 0)
    plsc.subcore_barrier()

    def out_body(m, carry):
        ch = m * _NS + s

        @pl.when(ch < nch)
        def _():
            pltpu.sync_copy(acc.at[pl.ds(ch * _ZR, _ZR)], zbuf)
            pltpu.sync_copy(zbuf, out_hbm.at[c, pl.ds(ch * _ZR, _ZR)])

        return carry

    lax.fori_loop(0, pl.cdiv(nch, _NS), out_body, 0)


def _make_deg_kernel(N, D, E, nc_pad):
    mesh = plsc.VectorSubcoreMesh(core_axis_name="c", subcore_axis_name="s")
    return pl.kernel(
        functools.partial(_deg_body, N, E),
        mesh=mesh,
        out_type=jax.ShapeDtypeStruct((_NC, N, D), jnp.float32),
        scratch_types=[
            pltpu.VMEM((nc_pad, _K), jnp.int32),
            pltpu.VMEM((_K, D), jnp.float32),
            pltpu.VMEM((_ZR, D), jnp.float32),
            pltpu.VMEM_SHARED((N, D), jnp.float32),
        ],
    )


def _agg_body(N, E, g_hbm, src_hbm, dst_hbm, zer_hbm, out_hbm,
              idx_s, dg0, dg1, rows0, rows1, zbuf, acc,
              sg0, sg1, sd0, sd1):
    c = lax.axis_index("c")
    s = lax.axis_index("s")
    wid = c * _NS + s
    nc = (E // _NW) // _K
    nfull = nc // _G
    rem = nc - nfull * _G
    assert nfull % 2 == 0 and rem % 2 == 0 and rem < _G

    pltpu.sync_copy(src_hbm.at[wid], idx_s)
    pltpu.async_copy(dst_hbm.at[wid, pl.ds(0, _G)], dg0, sd0)
    pltpu.async_copy(dst_hbm.at[wid, pl.ds(_G, _G)], dg1, sd1)

    pltpu.sync_copy(zer_hbm, zbuf)
    nch = N // _ZR

    def zc_body(m, carry):
        ch = m * _NS + s

        @pl.when(ch < nch)
        def _():
            pltpu.sync_copy(zbuf, acc.at[pl.ds(ch * _ZR, _ZR)])

        return carry

    lax.fori_loop(0, pl.cdiv(nch, _NS), zc_body, 0)
    plsc.subcore_barrier()

    pltpu.async_copy(g_hbm.at[idx_s.at[0]], rows0, sg0)
    pltpu.async_copy(g_hbm.at[idx_s.at[1]], rows1, sg1)

    def run_group(gbase, dg, sd, next_base):
        pltpu.make_async_copy(dst_hbm.at[wid, pl.ds(0, _G)], dg, sd).wait()

        def pair(l2, carry):
            for t, (rb, sgb) in ((0, (rows0, sg0)), (1, (rows1, sg1))):
                l = 2 * l2 + t
                j = gbase + l
                pltpu.make_async_copy(g_hbm.at[idx_s.at[0]], rb, sgb).wait()
                pltpu.sync_copy(rb, acc.at[dg.at[l]], add=True)

                @pl.when(j + 2 < nc)
                def _():
                    pltpu.async_copy(g_hbm.at[idx_s.at[j + 2]], rb, sgb)

            return carry

        lax.fori_loop(0, _G // 2, pair, 0)

        @pl.when(next_base < nc)
        def _():
            pltpu.async_copy(dst_hbm.at[wid, pl.ds(next_base, _G)], dg, sd)

    def groups(m, carry):
        gbase = (2 * m) * _G
        run_group(gbase, dg0, sd0, gbase + 2 * _G)
        run_group(gbase + _G, dg1, sd1, gbase + 3 * _G)
        return carry

    lax.fori_loop(0, nfull // 2, groups, 0)

    if rem:
        pltpu.make_async_copy(dst_hbm.at[wid, pl.ds(0, _G)], dg0, sd0).wait()
        for l in range(rem):
            j = nfull * _G + l
            rb, sgb = (rows0, sg0) if l % 2 == 0 else (rows1, sg1)
            pltpu.make_async_copy(g_hbm.at[idx_s.at[0]], rb, sgb).wait()
            pltpu.sync_copy(rb, acc.at[dg0.at[l]], add=True)
            if l + 2 < rem:
                pltpu.async_copy(g_hbm.at[idx_s.at[j + 2]], rb, sgb)
    plsc.subcore_barrier()

    def out_body(m, carry):
        ch = m * _NS + s

        @pl.when(ch < nch)
        def _():
            pltpu.sync_copy(acc.at[pl.ds(ch * _ZR, _ZR)], zbuf)
            pltpu.sync_copy(zbuf, out_hbm.at[c, pl.ds(ch * _ZR, _ZR)])

        return carry

    lax.fori_loop(0, pl.cdiv(nch, _NS), out_body, 0)


def _make_agg_kernel(N, D, E):
    n_chunks = (E // _NW) // _K
    mesh = plsc.VectorSubcoreMesh(core_axis_name="c", subcore_axis_name="s")
    return pl.kernel(
        functools.partial(_agg_body, N, E),
        mesh=mesh,
        out_type=jax.ShapeDtypeStruct((_NC, N, D), jnp.float32),
        scratch_types=[
            pltpu.VMEM((n_chunks, _K), jnp.int32),
            pltpu.VMEM((_G, _K), jnp.int32),
            pltpu.VMEM((_G, _K), jnp.int32),
            pltpu.VMEM((_K, D), jnp.float32),
            pltpu.VMEM((_K, D), jnp.float32),
            pltpu.VMEM((_ZR, D), jnp.float32),
            pltpu.VMEM_SHARED((N, D), jnp.float32),
            pltpu.SemaphoreType.DMA,
            pltpu.SemaphoreType.DMA,
            pltpu.SemaphoreType.DMA,
            pltpu.SemaphoreType.DMA,
        ],
    )


def _pre_body(betas, x_ref, w2_ref, degp_ref, g0_ref, dinv_ref, xw2_ref):
    x0 = jnp.maximum(x_ref[...], 0.0)
    deg = degp_ref[0][:, 0:1] + degp_ref[1][:, 0:1] + 1.0
    dinv = lax.rsqrt(deg)
    dinv_ref[...] = jnp.broadcast_to(dinv, x0.shape)
    g0_ref[...] = dinv * x0
    for i, b in enumerate(betas):
        m = jnp.dot(x0, w2_ref[i], preferred_element_type=jnp.float32)
        xw2_ref[i] = ALPHA * ((1.0 - b) * x0 + b * m)


def _make_pre(N, D, Lw, betas, R):
    return pl.pallas_call(
        functools.partial(_pre_body, betas),
        grid=(N // R,),
        in_specs=[
            pl.BlockSpec((R, D), lambda i: (i, 0)),
            pl.BlockSpec((Lw, D, D), lambda i: (0, 0, 0)),
            pl.BlockSpec((_NC, R, D), lambda i: (0, i, 0)),
        ],
        out_specs=[
            pl.BlockSpec((R, D), lambda i: (i, 0)),
            pl.BlockSpec((R, D), lambda i: (i, 0)),
            pl.BlockSpec((Lw, R, D), lambda i: (0, i, 0)),
        ],
        out_shape=[
            jax.ShapeDtypeStruct((N, D), jnp.float32),
            jax.ShapeDtypeStruct((N, D), jnp.float32),
            jax.ShapeDtypeStruct((Lw, N, D), jnp.float32),
        ],
    )


def _layer_body(beta, p_ref, g_ref, dinv_ref, xw2_ref, w1_ref, h_ref, g2_ref):
    dinv = dinv_ref[...]
    t = dinv * (p_ref[0] + p_ref[1] + g_ref[...])
    m = jnp.dot(t, w1_ref[...], preferred_element_type=jnp.float32)
    u = (1.0 - ALPHA) * ((1.0 - beta) * t + beta * m) + xw2_ref[...]
    h = jnp.maximum(u, 0.0)
    h_ref[...] = h
    g2_ref[...] = dinv * h


def _make_layer(N, D, beta, R):
    return pl.pallas_call(
        functools.partial(_layer_body, beta),
        grid=(N // R,),
        in_specs=[
            pl.BlockSpec((_NC, R, D), lambda i: (0, i, 0)),
            pl.BlockSpec((R, D), lambda i: (i, 0)),
            pl.BlockSpec((R, D), lambda i: (i, 0)),
            pl.BlockSpec((R, D), lambda i: (i, 0)),
            pl.BlockSpec((D, D), lambda i: (0, 0)),
        ],
        out_specs=[
            pl.BlockSpec((R, D), lambda i: (i, 0)),
            pl.BlockSpec((R, D), lambda i: (i, 0)),
        ],
        out_shape=[
            jax.ShapeDtypeStruct((N, D), jnp.float32),
            jax.ShapeDtypeStruct((N, D), jnp.float32),
        ],
    )


@jax.jit
def kernel(x, edge_index, W1, W2):
    N, D = x.shape
    E = edge_index.shape[1]
    Lw = W1.shape[0]
    betas = [math.log(THETA / (i + 1) + 1.0) for i in range(Lw)]
    R = 1000

    n_chunks = (E // _NW) // _K
    nc_pad = pl.cdiv(n_chunks, _G) * _G
    src3d = edge_index[0].reshape(_NW, n_chunks, _K)
    dst3d = edge_index[1].reshape(_NW, n_chunks, _K)
    dst3d = jnp.pad(dst3d, ((0, 0), (0, nc_pad - n_chunks), (0, 0)))
    zer = jnp.zeros((_ZR, D), jnp.float32)
    onesk = jnp.ones((_K, D), jnp.float32)

    degp = _make_deg_kernel(N, D, E, nc_pad)(dst3d, onesk, zer)
    g0, dinvb, xw2 = _make_pre(N, D, Lw, betas, R)(x, W2, degp)

    agg = _make_agg_kernel(N, D, E)
    g = g0
    h = None
    for i in range(Lw):
        part = agg(g, src3d, dst3d, zer)
        h, g = _make_layer(N, D, betas[i], R)(part, g, dinvb, xw2[i], W1[i])
    return h
